# baseline (speedup 1.0000x reference)
import jax
import jax.numpy as jnp
from jax.experimental import pallas as pl

N_STAGES = 4
K = 8192
D = 32
TILE = 256


def _rvq_kernel(x_ref, et_ref, out_ref):
    xt = x_ref[...]
    for q in range(N_STAGES):
        et = et_ref[q]
        e2 = jnp.sum(et * et, axis=0, keepdims=True)
        x2 = jnp.sum(xt * xt, axis=1, keepdims=True)
        s = jnp.dot(xt.astype(jnp.bfloat16), et.astype(jnp.bfloat16),
                    preferred_element_type=jnp.float32)
        dist = x2 - 2.0 * s + e2
        H = K // 2
        iota = jax.lax.broadcasted_iota(jnp.int32, (TILE, H), 1)
        d0 = dist[:, :H]
        d1 = dist[:, H:]
        m0 = jnp.min(d0, axis=1, keepdims=True)
        i0 = jnp.min(jnp.where(d0 == m0, iota, K), axis=1)
        m1 = jnp.min(d1, axis=1, keepdims=True)
        i1 = jnp.min(jnp.where(d1 == m1, iota + H, K), axis=1)
        m0b = m0.astype(jnp.bfloat16).astype(jnp.float32)
        idx = jnp.where((m1 < m0b)[:, 0], i1, i0)
        out_ref[q, :] = idx
        if q < N_STAGES - 1:
            iota_k = jax.lax.broadcasted_iota(jnp.int32, (TILE, K), 1)
            oh = (iota_k == idx[:, None]).astype(jnp.bfloat16)
            h1 = et.astype(jnp.bfloat16)
            r1 = et - h1.astype(jnp.float32)
            h2 = r1.astype(jnp.bfloat16)
            h3 = (r1 - h2.astype(jnp.float32)).astype(jnp.bfloat16)
            dims = (((1,), (1,)), ((), ()))
            qv = (jax.lax.dot_general(oh, h1, dims,
                                      preferred_element_type=jnp.float32)
                  + jax.lax.dot_general(oh, h2, dims,
                                        preferred_element_type=jnp.float32)
                  + jax.lax.dot_general(oh, h3, dims,
                                        preferred_element_type=jnp.float32))
            xt = xt - qv


def kernel(x, embed):
    b, d, t = x.shape
    n = b * t
    xf = jnp.transpose(x, (0, 2, 1)).reshape(n, d)
    et = jnp.transpose(embed, (0, 2, 1))
    out = pl.pallas_call(
        _rvq_kernel,
        grid=(n // TILE,),
        in_specs=[
            pl.BlockSpec((TILE, d), lambda i: (i, 0)),
            pl.BlockSpec((N_STAGES, d, K), lambda i: (0, 0, 0)),
        ],
        out_specs=pl.BlockSpec((N_STAGES, TILE), lambda i: (0, i)),
        out_shape=jax.ShapeDtypeStruct((N_STAGES, n), jnp.int32),
    )(xf, et)
    return jnp.transpose(out.reshape(N_STAGES, b, t), (1, 0, 2))


# speedup vs baseline: 2.4151x; 2.4151x over previous
import functools

import jax
import jax.numpy as jnp
from jax import lax
from jax.experimental import pallas as pl
from jax.experimental.pallas import tpu as pltpu
from jax.experimental.pallas import tpu_sc as plsc

N_STAGES = 4
K = 8192
D = 32
TILE = 256


def _stage_kernel(x_ref, et_ref, qv_refs, out_ref):
    xt = x_ref[...]
    for qv_ref in qv_refs:
        xt = xt - qv_ref[:, :D]
    et = et_ref[...]
    e2 = jnp.sum(et * et, axis=0, keepdims=True)
    x2 = jnp.sum(xt * xt, axis=1, keepdims=True)
    s = jnp.dot(xt.astype(jnp.bfloat16), et.astype(jnp.bfloat16),
                preferred_element_type=jnp.float32)
    dist = x2 - 2.0 * s + e2
    H = K // 2
    iota = lax.broadcasted_iota(jnp.int32, (TILE, H), 1)
    d0 = dist[:, :H]
    d1 = dist[:, H:]
    m0 = jnp.min(d0, axis=1, keepdims=True)
    i0 = jnp.min(jnp.where(d0 == m0, iota, K), axis=1)
    m1 = jnp.min(d1, axis=1, keepdims=True)
    i1 = jnp.min(jnp.where(d1 == m1, iota + H, K), axis=1)
    m0b = m0.astype(jnp.bfloat16).astype(jnp.float32)
    out_ref[0, :] = jnp.where((m1 < m0b)[:, 0], i1, i0)


def _tc_stage(n_prev, n, xf, et_q, qvs):
    body = lambda x_ref, et_ref, *rest: _stage_kernel(
        x_ref, et_ref, rest[:-1], rest[-1])
    out = pl.pallas_call(
        body,
        grid=(n // TILE,),
        in_specs=[pl.BlockSpec((TILE, D), lambda i: (i, 0)),
                  pl.BlockSpec((D, K), lambda i: (0, 0))]
        + [pl.BlockSpec((TILE, 128), lambda i: (i, 0))] * n_prev,
        out_specs=pl.BlockSpec((1, TILE), lambda i: (0, i)),
        out_shape=jax.ShapeDtypeStruct((1, n), jnp.int32),
    )(xf, et_q, *qvs)
    return out[0]


def _make_sc_gather(n):
    info = plsc.get_sparse_core_info()
    nw = info.num_cores * info.num_subcores
    bpw = n // nw
    mesh = plsc.VectorSubcoreMesh(core_axis_name="c", subcore_axis_name="s")

    @functools.partial(
        pl.kernel, mesh=mesh,
        out_type=jax.ShapeDtypeStruct((n, 128), jnp.float32),
        scratch_types=[
            pltpu.VMEM((bpw,), jnp.int32),
            pltpu.VMEM((bpw, 128), jnp.float32),
            pltpu.SemaphoreType.DMA,
        ],
    )
    def gather_rows(table_hbm, idx_hbm, out_hbm, idx_v, rows_v, sem):
        wid = lax.axis_index("s") * info.num_cores + lax.axis_index("c")
        base = wid * bpw
        pltpu.sync_copy(idx_hbm.at[pl.ds(base, bpw)], idx_v)
        pltpu.async_copy(table_hbm.at[idx_v], rows_v, sem).wait()
        pltpu.sync_copy(rows_v, out_hbm.at[pl.ds(base, bpw)])

    return gather_rows


def kernel(x, embed):
    b, d, t = x.shape
    n = b * t
    xf = jnp.transpose(x, (0, 2, 1)).reshape(n, d)
    et = jnp.transpose(embed, (0, 2, 1))
    sc_gather = _make_sc_gather(n)
    embed_pad = jnp.pad(embed, ((0, 0), (0, 0), (0, 128 - D)))
    idxs = []
    qvs = []
    for q in range(N_STAGES):
        idx_q = _tc_stage(q, n, xf, et[q], qvs)
        idxs.append(idx_q)
        if q < N_STAGES - 1:
            qvs.append(sc_gather(embed_pad[q], idx_q))
    out = jnp.stack(idxs, axis=0)
    return jnp.transpose(out.reshape(N_STAGES, b, t), (1, 0, 2))


# speedup vs baseline: 2.5435x; 1.0532x over previous
import functools

import jax
import jax.numpy as jnp
from jax import lax
from jax.experimental import pallas as pl
from jax.experimental.pallas import tpu as pltpu
from jax.experimental.pallas import tpu_sc as plsc

N_STAGES = 4
K = 8192
D = 32
TILE = 256


def _stage_kernel(x_ref, et_ref, qv_refs, out_ref):
    xt = x_ref[...]
    for qv_ref in qv_refs:
        xt = xt - qv_ref[:, :D]
    et = et_ref[...]
    e2 = jnp.sum(et * et, axis=0, keepdims=True)
    x2 = jnp.sum(xt * xt, axis=1, keepdims=True)
    s2 = jnp.dot(xt.astype(jnp.bfloat16), (et + et).astype(jnp.bfloat16),
                 preferred_element_type=jnp.float32)
    dist = (x2 - s2) + e2
    H = K // 2
    iota = lax.broadcasted_iota(jnp.int32, (TILE, H), 1)
    d0 = dist[:, :H]
    d1 = dist[:, H:]
    m0 = jnp.min(d0, axis=1, keepdims=True)
    m1 = jnp.min(d1, axis=1, keepdims=True)
    take = m1 < m0.astype(jnp.bfloat16).astype(jnp.float32)
    dwin = jnp.where(take, d1, d0)
    mwin = jnp.where(take, m1, m0)
    iw = jnp.min(jnp.where(dwin == mwin, iota, K), axis=1)
    out_ref[0, :] = iw + jnp.where(take[:, 0], H, 0)


def _tc_stage(n_prev, n, xf, et_q, qvs):
    body = lambda x_ref, et_ref, *rest: _stage_kernel(
        x_ref, et_ref, rest[:-1], rest[-1])
    out = pl.pallas_call(
        body,
        grid=(n // TILE,),
        in_specs=[pl.BlockSpec((TILE, D), lambda i: (i, 0)),
                  pl.BlockSpec((D, K), lambda i: (0, 0))]
        + [pl.BlockSpec((TILE, 128), lambda i: (i, 0))] * n_prev,
        out_specs=pl.BlockSpec((1, TILE), lambda i: (0, i)),
        out_shape=jax.ShapeDtypeStruct((1, n), jnp.int32),
    )(xf, et_q, *qvs)
    return out[0]


def _make_sc_gather(n):
    info = plsc.get_sparse_core_info()
    nw = info.num_cores * info.num_subcores
    bpw = n // nw
    mesh = plsc.VectorSubcoreMesh(core_axis_name="c", subcore_axis_name="s")

    @functools.partial(
        pl.kernel, mesh=mesh,
        out_type=jax.ShapeDtypeStruct((n, 128), jnp.float32),
        scratch_types=[
            pltpu.VMEM((bpw,), jnp.int32),
            pltpu.VMEM((bpw, 128), jnp.float32),
            pltpu.SemaphoreType.DMA,
        ],
    )
    def gather_rows(table_hbm, idx_hbm, out_hbm, idx_v, rows_v, sem):
        wid = lax.axis_index("s") * info.num_cores + lax.axis_index("c")
        base = wid * bpw
        pltpu.sync_copy(idx_hbm.at[pl.ds(base, bpw)], idx_v)
        pltpu.async_copy(table_hbm.at[idx_v], rows_v, sem).wait()
        pltpu.sync_copy(rows_v, out_hbm.at[pl.ds(base, bpw)])

    return gather_rows


def kernel(x, embed):
    b, d, t = x.shape
    n = b * t
    xf = jnp.transpose(x, (0, 2, 1)).reshape(n, d)
    et = jnp.transpose(embed, (0, 2, 1))
    sc_gather = _make_sc_gather(n)
    embed_pad = jnp.pad(embed, ((0, 0), (0, 0), (0, 128 - D)))
    idxs = []
    qvs = []
    for q in range(N_STAGES):
        idx_q = _tc_stage(q, n, xf, et[q], qvs)
        idxs.append(idx_q)
        if q < N_STAGES - 1:
            qvs.append(sc_gather(embed_pad[q], idx_q))
    out = jnp.stack(idxs, axis=0)
    return jnp.transpose(out.reshape(N_STAGES, b, t), (1, 0, 2))
